# baseline (speedup 1.0000x reference)
import functools

import jax
import jax.numpy as jnp
from jax import lax
from jax.experimental import pallas as pl
from jax.experimental.pallas import tpu as pltpu
from jax.experimental.pallas import tpu_sc as plsc

HID = 128
B = 16384
MAX_SLIP = 0.4
MAX_GUESS = 0.4
T = 50.0

NC, NS, L = 2, 16, 16
NW = NC * NS
BPW = B // NW
SIZES = (32, 96, 128, 128, 128)
OFFS = (0, 32, 128, 256, 384)
NCHUNK = len(SIZES)
KMAX = 128

_mesh = plsc.VectorSubcoreMesh(core_axis_name="c", subcore_axis_name="s")


def _sigmoid(x):
    return 1.0 / (1.0 + jnp.exp(-x))


@functools.partial(
    pl.kernel,
    mesh=_mesh,
    compiler_params=pltpu.CompilerParams(needs_layout_passes=False),
    out_type=jax.ShapeDtypeStruct((B,), jnp.float32),
    scratch_types=[
        pltpu.VMEM((BPW,), jnp.int32),
        pltpu.VMEM((BPW,), jnp.int32),
        pltpu.VMEM((SIZES[0], HID), jnp.float32),
        pltpu.VMEM((SIZES[1], HID), jnp.float32),
        pltpu.VMEM((SIZES[2], HID), jnp.float32),
        pltpu.VMEM((SIZES[3], HID), jnp.float32),
        pltpu.VMEM((SIZES[4], HID), jnp.float32),
        pltpu.VMEM((KMAX, HID), jnp.float32),
        pltpu.VMEM((KMAX, HID), jnp.float32),
        pltpu.VMEM((BPW,), jnp.float32),
        pltpu.VMEM((BPW,), jnp.float32),
        pltpu.VMEM((BPW,), jnp.float32),
        pltpu.VMEM((KMAX + L,), jnp.float32),
        pltpu.SemaphoreType.DMA,
        pltpu.SemaphoreType.DMA,
        pltpu.SemaphoreType.DMA,
        pltpu.SemaphoreType.DMA,
        pltpu.SemaphoreType.DMA,
        pltpu.SemaphoreType.DMA,
        pltpu.SemaphoreType.DMA,
        pltpu.SemaphoreType.DMA,
        pltpu.SemaphoreType.DMA,
        pltpu.SemaphoreType.DMA,
    ],
)
def _dina_sc(user_h, item_h, knowledge_h, theta_h, slip_h, guess_h, out_h,
             uidx, iidx, th_0, th_1, th_2, th_3, th_4, kn_a, kn_b,
             slipv, guessv, outv, nsum,
             sem_t0, sem_t1, sem_t2, sem_t3, sem_t4, sem_ka, sem_kb,
             sem_sg, sem_i, sem_o):
    wid = lax.axis_index("s") * NC + lax.axis_index("c")
    base = wid * BPW

    h_ui = pltpu.async_copy(user_h.at[pl.ds(base, BPW)], uidx, sem_i)
    h_ii = pltpu.async_copy(item_h.at[pl.ds(base, BPW)], iidx, sem_i)

    kn_bufs = (kn_a, kn_b)
    kn_sems = (sem_ka, sem_kb)

    def start_kn(c):
        return pltpu.async_copy(
            knowledge_h.at[pl.ds(base + OFFS[c], SIZES[c])],
            kn_bufs[c % 2].at[pl.ds(0, SIZES[c])], kn_sems[c % 2])

    th_bufs = (th_0, th_1, th_2, th_3, th_4)
    th_sems = (sem_t0, sem_t1, sem_t2, sem_t3, sem_t4)

    def start(c):
        return pltpu.async_copy(
            theta_h.at[uidx.at[pl.ds(OFFS[c], SIZES[c])]], th_bufs[c],
            th_sems[c])

    h_ui.wait()
    handles = [start(c) for c in range(NCHUNK)]
    kn_handles = [None] * NCHUNK
    kn_handles[0] = start_kn(0)
    kn_handles[1] = start_kn(1)

    h_ii.wait()
    sg_handles = []
    for c in range(NCHUNK):
        sg_handles.append(pltpu.async_copy(
            slip_h.at[iidx.at[pl.ds(OFFS[c], SIZES[c])]],
            slipv.at[pl.ds(OFFS[c], SIZES[c])], sem_sg))
        sg_handles.append(pltpu.async_copy(
            guess_h.at[iidx.at[pl.ds(OFFS[c], SIZES[c])]],
            guessv.at[pl.ds(OFFS[c], SIZES[c])], sem_sg))

    iota = lax.iota(jnp.int32, L)
    mask_last = iota == (L - 1)
    out_handles = []

    for c in range(NCHUNK):
        handles[c].wait()
        kn_handles[c].wait()
        th = th_bufs[c]
        kn = kn_bufs[c % 2]

        @plsc.parallel_loop(0, SIZES[c], unroll=4)
        def row_body(r):
            parts = []
            for k in range(HID // L):
                th_v = th[r, pl.ds(k * L, L)]
                kn_v = kn[r, pl.ds(k * L, L)]
                q = _sigmoid(th_v)
                parts.append((q - 0.5) * kn_v)
            s = (((parts[0] + parts[1]) + (parts[2] + parts[3]))
                 + ((parts[4] + parts[5]) + (parts[6] + parts[7])))
            cs = plsc.cumsum(s)
            plsc.store_compressed(nsum.at[pl.ds(r, L)], cs, mask=mask_last)

        if c + 2 < NCHUNK:
            kn_handles[c + 2] = start_kn(c + 2)
        if c == 0:
            for h in sg_handles:
                h.wait()

        @plsc.parallel_loop(0, SIZES[c], step=L)
        def score_body(goff):
            off = OFFS[c] + goff
            n = nsum[pl.ds(goff, L)]
            z = n * (1.0 / T)
            p = _sigmoid(z)
            slip = MAX_SLIP * _sigmoid(slipv[pl.ds(off, L)])
            guess = MAX_GUESS * _sigmoid(guessv[pl.ds(off, L)])
            outv[pl.ds(off, L)] = guess + (1.0 - slip - guess) * p

        out_handles.append(pltpu.async_copy(
            outv.at[pl.ds(OFFS[c], SIZES[c])],
            out_h.at[pl.ds(base + OFFS[c], SIZES[c])], sem_o))

    for h in out_handles:
        h.wait()


def kernel(user, item, knowledge, theta_table, slip_table, guess_table):
    user = user.astype(jnp.int32)
    item = item.astype(jnp.int32)
    slip_flat = slip_table.reshape((-1,))
    guess_flat = guess_table.reshape((-1,))
    return _dina_sc(user, item, knowledge, theta_table, slip_flat, guess_flat)


# speedup vs baseline: 1.0054x; 1.0054x over previous
import functools

import jax
import jax.numpy as jnp
from jax import lax
from jax.experimental import pallas as pl
from jax.experimental.pallas import tpu as pltpu
from jax.experimental.pallas import tpu_sc as plsc

HID = 128
B = 16384
MAX_SLIP = 0.4
MAX_GUESS = 0.4
T = 50.0

NC, NS, L = 2, 16, 16
NW = NC * NS
BPW = B // NW
CHUNK = 128
NCHUNK = BPW // CHUNK

_mesh = plsc.VectorSubcoreMesh(core_axis_name="c", subcore_axis_name="s")


def _sigmoid_fast(x):
    return 1.0 / (1.0 + jnp.exp(-x))


@functools.partial(
    pl.kernel,
    mesh=_mesh,
    compiler_params=pltpu.CompilerParams(needs_layout_passes=False),
    out_type=jax.ShapeDtypeStruct((B,), jnp.float32),
    scratch_types=[
        pltpu.VMEM((NCHUNK, CHUNK), jnp.int32),
        pltpu.VMEM((NCHUNK, CHUNK), jnp.int32),
        pltpu.VMEM((CHUNK, HID), jnp.float32),
        pltpu.VMEM((CHUNK, HID), jnp.float32),
        pltpu.VMEM((CHUNK, HID), jnp.float32),
        pltpu.VMEM((CHUNK, HID), jnp.float32),
        pltpu.VMEM((CHUNK, HID), jnp.float32),
        pltpu.VMEM((CHUNK, HID), jnp.float32),
        pltpu.VMEM((BPW,), jnp.float32),
        pltpu.VMEM((BPW,), jnp.float32),
        pltpu.VMEM((BPW,), jnp.float32),
        pltpu.VMEM((CHUNK + L,), jnp.float32),
        pltpu.SemaphoreType.DMA,
        pltpu.SemaphoreType.DMA,
        pltpu.SemaphoreType.DMA,
        pltpu.SemaphoreType.DMA,
        pltpu.SemaphoreType.DMA,
        pltpu.SemaphoreType.DMA,
        pltpu.SemaphoreType.DMA,
        pltpu.SemaphoreType.DMA,
        pltpu.SemaphoreType.DMA,
    ],
)
def _dina_sc(user_h, item_h, knowledge_h, theta_h, slip_h, guess_h, out_h,
             uidx, iidx, th_a, th_b, th_c, th_d, kn_a, kn_b,
             slipv, guessv, outv, nsum,
             sem_ta, sem_tb, sem_tc, sem_td, sem_ka, sem_kb,
             sem_sg, sem_i, sem_o):
    wid = lax.axis_index("s") * NC + lax.axis_index("c")
    base = wid * BPW

    idx_handles = []
    for c in range(NCHUNK):
        idx_handles.append(pltpu.async_copy(
            user_h.at[pl.ds(base + c * CHUNK, CHUNK)], uidx.at[c], sem_i))
        idx_handles.append(pltpu.async_copy(
            item_h.at[pl.ds(base + c * CHUNK, CHUNK)], iidx.at[c], sem_i))

    kn_bufs = (kn_a, kn_b)
    kn_sems = (sem_ka, sem_kb)

    def start_kn(c):
        return pltpu.async_copy(
            knowledge_h.at[pl.ds(base + c * CHUNK, CHUNK)],
            kn_bufs[c % 2], kn_sems[c % 2])

    for h in idx_handles:
        h.wait()

    th_bufs = (th_a, th_b, th_c, th_d)
    th_sems = (sem_ta, sem_tb, sem_tc, sem_td)

    def start(c):
        return pltpu.async_copy(theta_h.at[uidx.at[c]], th_bufs[c],
                                th_sems[c])

    handles = [start(c) for c in range(NCHUNK)]
    kn_handles = [None] * NCHUNK
    kn_handles[0] = start_kn(0)
    kn_handles[1] = start_kn(1)

    sg_handles = []
    for c in range(NCHUNK):
        sg_handles.append(pltpu.async_copy(
            slip_h.at[iidx.at[c]], slipv.at[pl.ds(c * CHUNK, CHUNK)], sem_sg))
        sg_handles.append(pltpu.async_copy(
            guess_h.at[iidx.at[c]], guessv.at[pl.ds(c * CHUNK, CHUNK)], sem_sg))

    iota = lax.iota(jnp.int32, L)
    mask_last = iota == (L - 1)
    out_handles = []

    for c in range(NCHUNK):
        handles[c].wait()
        kn_handles[c].wait()
        th = th_bufs[c]
        kn = kn_bufs[c % 2]

        @plsc.parallel_loop(0, CHUNK, unroll=4)
        def row_body(r):
            parts = []
            for k in range(HID // L):
                th_v = th[r, pl.ds(k * L, L)]
                kn_v = kn[r, pl.ds(k * L, L)]
                q = _sigmoid_fast(th_v)
                parts.append((q - 0.5) * kn_v)
            s = (((parts[0] + parts[1]) + (parts[2] + parts[3]))
                 + ((parts[4] + parts[5]) + (parts[6] + parts[7])))
            cs = plsc.cumsum(s)
            plsc.store_compressed(nsum.at[pl.ds(r, L)], cs, mask=mask_last)

        if c + 2 < NCHUNK:
            kn_handles[c + 2] = start_kn(c + 2)
        if c == 0:
            for h in sg_handles:
                h.wait()

        @plsc.parallel_loop(0, CHUNK, step=L)
        def score_body(goff):
            off = c * CHUNK + goff
            n = nsum[pl.ds(goff, L)]
            z = n * (1.0 / T)
            p = _sigmoid_fast(z)
            slip = MAX_SLIP * _sigmoid_fast(slipv[pl.ds(off, L)])
            guess = MAX_GUESS * _sigmoid_fast(guessv[pl.ds(off, L)])
            outv[pl.ds(off, L)] = guess + (1.0 - slip - guess) * p

        out_handles.append(pltpu.async_copy(
            outv.at[pl.ds(c * CHUNK, CHUNK)],
            out_h.at[pl.ds(base + c * CHUNK, CHUNK)], sem_o))

    for h in out_handles:
        h.wait()


def kernel(user, item, knowledge, theta_table, slip_table, guess_table):
    user = user.astype(jnp.int32)
    item = item.astype(jnp.int32)
    slip_flat = slip_table.reshape((-1,))
    guess_flat = guess_table.reshape((-1,))
    return _dina_sc(user, item, knowledge, theta_table, slip_flat, guess_flat)
